# baseline (speedup 1.0000x reference)
import functools

import jax
import jax.numpy as jnp
from jax import lax
from jax.experimental import pallas as pl
from jax.experimental.pallas import tpu as pltpu
from jax.experimental.pallas import tpu_sc as plsc

N = 10000
E = 320000
D = 128
DE = 16
EPS = 1e-5

NC = 2
NS = 16
NW = NC * NS
EP = E // NW
C = 80
EPP = 10240
NCHUNK = EPP // C
E_PAD = NW * EPP
NPAD = 10240
ROWS_PER_TILE = NPAD // NS

_EB = 2560


def _emb_body(ea_ref, we0_ref, we1_ref, we2_ref, e0_ref, e1_ref, e2_ref):
    ea = ea_ref[...]
    e0_ref[...] = jnp.dot(ea, we0_ref[...], preferred_element_type=jnp.float32)
    e1_ref[...] = jnp.dot(ea, we1_ref[...], preferred_element_type=jnp.float32)
    e2_ref[...] = jnp.dot(ea, we2_ref[...], preferred_element_type=jnp.float32)


def _edge_embeddings(edge_attr_pad, We0, We1, We2):
    grid = (E_PAD // _EB,)
    eb_spec = pl.BlockSpec((_EB, DE), lambda i: (i, 0))
    w_spec = pl.BlockSpec((DE, D), lambda i: (0, 0))
    out_spec = pl.BlockSpec((_EB, D), lambda i: (i, 0))
    return pl.pallas_call(
        _emb_body,
        grid=grid,
        in_specs=[eb_spec, w_spec, w_spec, w_spec],
        out_specs=[out_spec, out_spec, out_spec],
        out_shape=[jax.ShapeDtypeStruct((E_PAD, D), jnp.float32)] * 3,
    )(edge_attr_pad, We0, We1, We2)


_sc_mesh = plsc.VectorSubcoreMesh(core_axis_name="c", subcore_axis_name="s")


@functools.partial(
    pl.kernel,
    out_type=jax.ShapeDtypeStruct((NC, NPAD, D), jnp.float32),
    mesh=_sc_mesh,
    scratch_types=[
        pltpu.VMEM_SHARED((NPAD, D), jnp.float32),
        pltpu.VMEM((4, C), jnp.int32),
        pltpu.VMEM((4, C), jnp.int32),
        pltpu.VMEM((2, C, D), jnp.float32),
        pltpu.VMEM((2, C, D), jnp.float32),
        pltpu.SemaphoreType.DMA,
        pltpu.SemaphoreType.DMA,
        pltpu.SemaphoreType.DMA,
        pltpu.SemaphoreType.DMA,
    ],
)
def _sc_aggregate(x_hbm, src_hbm, dst_hbm, emb_hbm, zero_hbm, out_hbm,
                  agg, srcv, dstv, xg, ev, isem, gsem, esem, ssem):
    cid = lax.axis_index("c")
    sid = lax.axis_index("s")
    wid = sid * NC + cid
    e0 = wid * EPP

    pltpu.sync_copy(src_hbm.at[pl.ds(e0, C)], srcv.at[0])
    pltpu.sync_copy(dst_hbm.at[pl.ds(e0, C)], dstv.at[0])
    pltpu.sync_copy(src_hbm.at[pl.ds(e0 + C, C)], srcv.at[1])
    pltpu.sync_copy(dst_hbm.at[pl.ds(e0 + C, C)], dstv.at[1])
    pltpu.async_copy(x_hbm.at[srcv.at[0]], xg.at[0], gsem)
    pltpu.async_copy(emb_hbm.at[pl.ds(e0, C), :], ev.at[0], esem)

    r0 = sid * ROWS_PER_TILE
    pltpu.sync_copy(zero_hbm.at[pl.ds(r0, ROWS_PER_TILE), :],
                    agg.at[pl.ds(r0, ROWS_PER_TILE), :])
    plsc.subcore_barrier()

    def _chunk(k, _):
        p = k & 1
        pn = 1 - p
        pltpu.make_async_copy(x_hbm.at[srcv.at[0]], xg.at[p], gsem).wait()
        pltpu.make_async_copy(emb_hbm.at[pl.ds(e0, C), :], ev.at[p], esem).wait()
        @pl.when(k > 0)
        def _():
            pltpu.make_async_copy(xg.at[pn], agg.at[dstv.at[0]], ssem).wait()
        @pl.when(jnp.logical_and(k > 0, k + 1 < NCHUNK))
        def _():
            pltpu.make_async_copy(src_hbm.at[pl.ds(e0, C)], srcv.at[0], isem).wait()
            pltpu.make_async_copy(dst_hbm.at[pl.ds(e0, C)], dstv.at[0], isem).wait()
        @pl.when(k + 1 < NCHUNK)
        def _():
            nb = (k + 1) * C
            pltpu.async_copy(x_hbm.at[srcv.at[(k + 1) & 3]], xg.at[pn], gsem)
            pltpu.async_copy(emb_hbm.at[pl.ds(e0 + nb, C), :], ev.at[pn], esem)
        @pl.when(k + 2 < NCHUNK)
        def _():
            nb2 = e0 + (k + 2) * C
            s2 = (k + 2) & 3
            pltpu.async_copy(src_hbm.at[pl.ds(nb2, C)], srcv.at[s2], isem)
            pltpu.async_copy(dst_hbm.at[pl.ds(nb2, C)], dstv.at[s2], isem)

        def _row(i, _):
            for j in range(D // 16):
                sl = pl.ds(j * 16, 16)
                xg[p, i, sl] = jnp.maximum(xg[p, i, sl] + ev[p, i, sl], 0.0)
            return 0

        lax.fori_loop(0, C, _row, 0)
        pltpu.async_copy(xg.at[p], agg.at[dstv.at[k & 3]], ssem, add=True)
        return 0

    lax.fori_loop(0, NCHUNK, _chunk, 0)
    pltpu.make_async_copy(xg.at[0], agg.at[dstv.at[0]], ssem).wait()
    plsc.subcore_barrier()

    pltpu.sync_copy(agg.at[pl.ds(r0, ROWS_PER_TILE), :],
                    out_hbm.at[cid, pl.ds(r0, ROWS_PER_TILE), :])


_NB = 1000
_NBLK = N // _NB


def _dense_body(x_ref, p0_ref, p1_ref, w_ref, h_ref, st_ref, sum_ref, sq_ref):
    i = pl.program_id(0)
    y = x_ref[...] + p0_ref[0] + p1_ref[0]
    h = jnp.maximum(jnp.dot(y, w_ref[...], preferred_element_type=jnp.float32), 0.0)
    h_ref[...] = h

    @pl.when(i == 0)
    def _():
        sum_ref[...] = jnp.zeros_like(sum_ref)
        sq_ref[...] = jnp.zeros_like(sq_ref)

    sum_ref[...] += jnp.sum(h, axis=0, keepdims=True)
    sq_ref[...] += jnp.sum(h * h, axis=0, keepdims=True)
    st_ref[0:1, :] = sum_ref[...]
    st_ref[1:2, :] = sq_ref[...]


def _dense(x, partial, W):
    grid = (_NBLK,)
    x_spec = pl.BlockSpec((_NB, D), lambda i: (i, 0))
    p0_spec = pl.BlockSpec((1, _NB, D), lambda i: (0, i, 0))
    p1_spec = pl.BlockSpec((1, _NB, D), lambda i: (1, i, 0))
    w_spec = pl.BlockSpec((D, D), lambda i: (0, 0))
    h_spec = pl.BlockSpec((_NB, D), lambda i: (i, 0))
    st_spec = pl.BlockSpec((2, D), lambda i: (0, 0))
    return pl.pallas_call(
        _dense_body,
        grid=grid,
        in_specs=[x_spec, p0_spec, p1_spec, w_spec],
        out_specs=[h_spec, st_spec],
        out_shape=[
            jax.ShapeDtypeStruct((N, D), jnp.float32),
            jax.ShapeDtypeStruct((2, D), jnp.float32),
        ],
        scratch_shapes=[
            pltpu.VMEM((1, D), jnp.float32),
            pltpu.VMEM((1, D), jnp.float32),
        ],
    )(x, partial, partial, W)


def _bn_body(h_ref, x_ref, st_ref, g_ref, b_ref, o_ref):
    inv_n = jnp.float32(1.0 / N)
    mu = st_ref[0:1, :] * inv_n
    var = st_ref[1:2, :] * inv_n - mu * mu
    inv = lax.rsqrt(var + EPS)
    o_ref[...] = (h_ref[...] - mu) * inv * g_ref[...] + b_ref[...] + x_ref[...]


def _bn_residual(h, x, stats, gamma, beta):
    grid = (_NBLK,)
    blk = pl.BlockSpec((_NB, D), lambda i: (i, 0))
    row = pl.BlockSpec((1, D), lambda i: (0, 0))
    st_spec = pl.BlockSpec((2, D), lambda i: (0, 0))
    return pl.pallas_call(
        _bn_body,
        grid=grid,
        in_specs=[blk, blk, st_spec, row, row],
        out_specs=blk,
        out_shape=jax.ShapeDtypeStruct((N, D), jnp.float32),
    )(h, x, stats, gamma, beta)


def kernel(x, edge_index, edge_attr, batch,
           We0, W0, gamma0, beta0,
           We1, W1, gamma1, beta1,
           We2, W2, gamma2, beta2):
    del batch
    pad = EPP - EP
    src = jnp.pad(edge_index[0].reshape(NW, EP), ((0, 0), (0, pad))).reshape(-1)
    dst = jnp.pad(edge_index[1].reshape(NW, EP), ((0, 0), (0, pad)),
                  constant_values=N).reshape(-1)
    ea_pad = jnp.pad(edge_attr.reshape(NW, EP, DE),
                     ((0, 0), (0, pad), (0, 0))).reshape(-1, DE)
    zeros = jnp.zeros((NPAD, D), jnp.float32)
    embs = _edge_embeddings(ea_pad, We0, We1, We2)
    params = [(W0, gamma0, beta0), (W1, gamma1, beta1), (W2, gamma2, beta2)]
    for emb, (W, g, b) in zip(embs, params):
        partial = _sc_aggregate(x, src, dst, emb, zeros)
        h, stats = _dense(x, partial, W)
        x = _bn_residual(h, x, stats, g.reshape(1, D), b.reshape(1, D))
    return x


# speedup vs baseline: 1.6137x; 1.6137x over previous
import functools

import jax
import jax.numpy as jnp
from jax import lax
from jax.experimental import pallas as pl
from jax.experimental.pallas import tpu as pltpu
from jax.experimental.pallas import tpu_sc as plsc

N = 10000
E = 320000
D = 128
DE = 16
EPS = 1e-5

NC = 2
NS = 16
NW = NC * NS
EP = E // NW
C = 80
EPP = 10240
NCHUNK = EPP // C
E_PAD = NW * EPP
NPAD = 10240
ROWS_PER_TILE = NPAD // NS

_EB = 2560


def _emb_body(ea_ref, we0_ref, we1_ref, we2_ref, e0_ref, e1_ref, e2_ref):
    ea = ea_ref[...]
    e0_ref[...] = jnp.dot(ea, we0_ref[...], preferred_element_type=jnp.float32)
    e1_ref[...] = jnp.dot(ea, we1_ref[...], preferred_element_type=jnp.float32)
    e2_ref[...] = jnp.dot(ea, we2_ref[...], preferred_element_type=jnp.float32)


def _edge_embeddings(edge_attr_pad, We0, We1, We2):
    grid = (E_PAD // _EB,)
    eb_spec = pl.BlockSpec((_EB, DE), lambda i: (i, 0))
    w_spec = pl.BlockSpec((DE, D), lambda i: (0, 0))
    out_spec = pl.BlockSpec((_EB, D), lambda i: (i, 0))
    return pl.pallas_call(
        _emb_body,
        grid=grid,
        in_specs=[eb_spec, w_spec, w_spec, w_spec],
        out_specs=[out_spec, out_spec, out_spec],
        out_shape=[jax.ShapeDtypeStruct((E_PAD, D), jnp.float32)] * 3,
    )(edge_attr_pad, We0, We1, We2)


_sc_mesh = plsc.VectorSubcoreMesh(core_axis_name="c", subcore_axis_name="s")


@functools.partial(
    pl.kernel,
    out_type=jax.ShapeDtypeStruct((NC, NPAD, D), jnp.float32),
    mesh=_sc_mesh,
    scratch_types=[
        pltpu.VMEM_SHARED((NPAD, D), jnp.float32),
        pltpu.VMEM((2, 2, 128), jnp.int32),
        pltpu.VMEM((2, C), jnp.int32),
        pltpu.VMEM((2, C, D), jnp.float32),
        pltpu.VMEM((2, C, D), jnp.float32),
        pltpu.SemaphoreType.DMA,
        pltpu.SemaphoreType.DMA,
        pltpu.SemaphoreType.DMA,
        pltpu.SemaphoreType.DMA,
        pltpu.SemaphoreType.DMA,
        pltpu.SemaphoreType.DMA,
        pltpu.SemaphoreType.DMA,
        pltpu.SemaphoreType.DMA,
    ],
)
def _sc_aggregate(x_hbm, idx_hbm, emb_hbm, zero_hbm, out_hbm,
                  agg, idxv, dstc, xg, ev,
                  isem0, isem1, gsem0, gsem1, esem0, esem1, ssem0, ssem1):
    cid = lax.axis_index("c")
    sid = lax.axis_index("s")
    wid = sid * NC + cid
    e0 = wid * EPP
    g0 = wid * NCHUNK
    isem = (isem0, isem1)
    gsem = (gsem0, gsem1)
    esem = (esem0, esem1)
    ssem = (ssem0, ssem1)

    def _copy_dst_row(s):
        for j in range(C // 16):
            dstc[s, pl.ds(j * 16, 16)] = idxv[s, 1, pl.ds(j * 16, 16)]

    pltpu.sync_copy(idx_hbm.at[g0], idxv.at[0])
    pltpu.sync_copy(idx_hbm.at[g0 + 1], idxv.at[1])
    _copy_dst_row(0)
    _copy_dst_row(1)
    pltpu.async_copy(x_hbm.at[idxv.at[0, 0, pl.ds(0, C)]], xg.at[0], gsem[0])
    pltpu.async_copy(emb_hbm.at[pl.ds(e0, C), :], ev.at[0], esem[0])

    r0 = sid * ROWS_PER_TILE
    pltpu.sync_copy(zero_hbm.at[pl.ds(r0, ROWS_PER_TILE), :],
                    agg.at[pl.ds(r0, ROWS_PER_TILE), :])
    plsc.subcore_barrier()

    def _pair(jj, _):
        for b in (0, 1):
            k = 2 * jj + b
            nb = 1 - b
            @pl.when(k > 0)
            def _():
                pltpu.make_async_copy(
                    xg.at[nb], agg.at[dstc.at[nb]], ssem[nb]).wait()
            @pl.when(jnp.logical_and(k > 0, k + 1 < NCHUNK))
            def _():
                pltpu.make_async_copy(
                    idx_hbm.at[g0], idxv.at[nb], isem[nb]).wait()
            @pl.when(k + 1 < NCHUNK)
            def _():
                pltpu.async_copy(
                    x_hbm.at[idxv.at[nb, 0, pl.ds(0, C)]], xg.at[nb], gsem[nb])
                pltpu.async_copy(
                    emb_hbm.at[pl.ds(e0 + (k + 1) * C, C), :], ev.at[nb], esem[nb])
            @pl.when(jnp.logical_and(k > 0, k + 1 < NCHUNK))
            def _():
                _copy_dst_row(nb)
            @pl.when(k + 2 < NCHUNK)
            def _():
                pltpu.async_copy(idx_hbm.at[g0 + k + 2], idxv.at[b], isem[b])
            pltpu.make_async_copy(
                x_hbm.at[idxv.at[b, 0, pl.ds(0, C)]], xg.at[b], gsem[b]).wait()
            pltpu.make_async_copy(
                emb_hbm.at[pl.ds(e0, C), :], ev.at[b], esem[b]).wait()

            def _row(i, _):
                for j in range(D // 16):
                    sl = pl.ds(j * 16, 16)
                    xg[b, i, sl] = jnp.maximum(xg[b, i, sl] + ev[b, i, sl], 0.0)
                return 0

            lax.fori_loop(0, C, _row, 0)
            pltpu.async_copy(xg.at[b], agg.at[dstc.at[b]], ssem[b], add=True)
        return 0

    lax.fori_loop(0, NCHUNK // 2, _pair, 0)
    pltpu.make_async_copy(xg.at[1], agg.at[dstc.at[1]], ssem[1]).wait()
    plsc.subcore_barrier()

    pltpu.sync_copy(agg.at[pl.ds(r0, ROWS_PER_TILE), :],
                    out_hbm.at[cid, pl.ds(r0, ROWS_PER_TILE), :])


_NB = 1000
_NBLK = N // _NB


def _dense_body(x_ref, p0_ref, p1_ref, w_ref, h_ref, st_ref, sum_ref, sq_ref):
    i = pl.program_id(0)
    y = x_ref[...] + p0_ref[0] + p1_ref[0]
    h = jnp.maximum(jnp.dot(y, w_ref[...], preferred_element_type=jnp.float32), 0.0)
    h_ref[...] = h

    @pl.when(i == 0)
    def _():
        sum_ref[...] = jnp.zeros_like(sum_ref)
        sq_ref[...] = jnp.zeros_like(sq_ref)

    sum_ref[...] += jnp.sum(h, axis=0, keepdims=True)
    sq_ref[...] += jnp.sum(h * h, axis=0, keepdims=True)
    st_ref[0:1, :] = sum_ref[...]
    st_ref[1:2, :] = sq_ref[...]


def _dense(x, partial, W):
    grid = (_NBLK,)
    x_spec = pl.BlockSpec((_NB, D), lambda i: (i, 0))
    p0_spec = pl.BlockSpec((1, _NB, D), lambda i: (0, i, 0))
    p1_spec = pl.BlockSpec((1, _NB, D), lambda i: (1, i, 0))
    w_spec = pl.BlockSpec((D, D), lambda i: (0, 0))
    h_spec = pl.BlockSpec((_NB, D), lambda i: (i, 0))
    st_spec = pl.BlockSpec((2, D), lambda i: (0, 0))
    return pl.pallas_call(
        _dense_body,
        grid=grid,
        in_specs=[x_spec, p0_spec, p1_spec, w_spec],
        out_specs=[h_spec, st_spec],
        out_shape=[
            jax.ShapeDtypeStruct((N, D), jnp.float32),
            jax.ShapeDtypeStruct((2, D), jnp.float32),
        ],
        scratch_shapes=[
            pltpu.VMEM((1, D), jnp.float32),
            pltpu.VMEM((1, D), jnp.float32),
        ],
    )(x, partial, partial, W)


def _bn_body(h_ref, x_ref, st_ref, g_ref, b_ref, o_ref):
    inv_n = jnp.float32(1.0 / N)
    mu = st_ref[0:1, :] * inv_n
    var = st_ref[1:2, :] * inv_n - mu * mu
    inv = lax.rsqrt(var + EPS)
    o_ref[...] = (h_ref[...] - mu) * inv * g_ref[...] + b_ref[...] + x_ref[...]


def _bn_residual(h, x, stats, gamma, beta):
    grid = (_NBLK,)
    blk = pl.BlockSpec((_NB, D), lambda i: (i, 0))
    row = pl.BlockSpec((1, D), lambda i: (0, 0))
    st_spec = pl.BlockSpec((2, D), lambda i: (0, 0))
    return pl.pallas_call(
        _bn_body,
        grid=grid,
        in_specs=[blk, blk, st_spec, row, row],
        out_specs=blk,
        out_shape=jax.ShapeDtypeStruct((N, D), jnp.float32),
    )(h, x, stats, gamma, beta)


def kernel(x, edge_index, edge_attr, batch,
           We0, W0, gamma0, beta0,
           We1, W1, gamma1, beta1,
           We2, W2, gamma2, beta2):
    del batch
    pad = EPP - EP
    src = jnp.pad(edge_index[0].reshape(NW, EP), ((0, 0), (0, pad)))
    dst = jnp.pad(edge_index[1].reshape(NW, EP), ((0, 0), (0, pad)),
                  constant_values=N)
    s3 = jnp.pad(src.reshape(NW, NCHUNK, C), ((0, 0), (0, 0), (0, 128 - C)))
    d3 = jnp.pad(dst.reshape(NW, NCHUNK, C), ((0, 0), (0, 0), (0, 128 - C)),
                 constant_values=N)
    idx = jnp.stack([s3, d3], axis=2).reshape(NW * NCHUNK, 2, 128)
    ea_pad = jnp.pad(edge_attr.reshape(NW, EP, DE),
                     ((0, 0), (0, pad), (0, 0))).reshape(-1, DE)
    zeros = jnp.zeros((NPAD, D), jnp.float32)
    embs = _edge_embeddings(ea_pad, We0, We1, We2)
    params = [(W0, gamma0, beta0), (W1, gamma1, beta1), (W2, gamma2, beta2)]
    for emb, (W, g, b) in zip(embs, params):
        partial = _sc_aggregate(x, idx, emb, zeros)
        h, stats = _dense(x, partial, W)
        x = _bn_residual(h, x, stats, g.reshape(1, D), b.reshape(1, D))
    return x


# speedup vs baseline: 1.6343x; 1.0127x over previous
import functools

import jax
import jax.numpy as jnp
from jax import lax
from jax.experimental import pallas as pl
from jax.experimental.pallas import tpu as pltpu
from jax.experimental.pallas import tpu_sc as plsc

N = 10000
E = 320000
D = 128
DE = 16
EPS = 1e-5

NC = 2
NS = 16
NW = NC * NS
EP = E // NW
C = 80
EPP = 10240
NCHUNK = EPP // C
E_PAD = NW * EPP
G = NW * NCHUNK
NPAD = 10240
ROWS_PER_TILE = NPAD // NS

_EB = 2560


def _emb_body(ea_ref, we0_ref, we1_ref, we2_ref, e0_ref, e1_ref, e2_ref):
    ea = ea_ref[...]
    e0_ref[...] = jnp.dot(ea, we0_ref[...], preferred_element_type=jnp.float32)
    e1_ref[...] = jnp.dot(ea, we1_ref[...], preferred_element_type=jnp.float32)
    e2_ref[...] = jnp.dot(ea, we2_ref[...], preferred_element_type=jnp.float32)


def _edge_embeddings(edge_attr_pad, We0, We1, We2):
    grid = (E_PAD // _EB,)
    eb_spec = pl.BlockSpec((_EB, DE), lambda i: (i, 0))
    w_spec = pl.BlockSpec((DE, D), lambda i: (0, 0))
    out_spec = pl.BlockSpec((_EB, D), lambda i: (i, 0))
    return pl.pallas_call(
        _emb_body,
        grid=grid,
        in_specs=[eb_spec, w_spec, w_spec, w_spec],
        out_specs=[out_spec, out_spec, out_spec],
        out_shape=[jax.ShapeDtypeStruct((E_PAD, D), jnp.float32)] * 3,
    )(edge_attr_pad, We0, We1, We2)


_sc_mesh = plsc.VectorSubcoreMesh(core_axis_name="c", subcore_axis_name="s")


@functools.partial(
    pl.kernel,
    out_type=jax.ShapeDtypeStruct((NC, NPAD, D), jnp.float32),
    mesh=_sc_mesh,
    scratch_types=[
        pltpu.VMEM_SHARED((NPAD, D), jnp.float32),
        pltpu.VMEM((2, 2, 128), jnp.int32),
        pltpu.VMEM((2, C), jnp.int32),
        pltpu.VMEM((2, C, D), jnp.float32),
        pltpu.VMEM((2, C, D), jnp.float32),
        pltpu.SemaphoreType.DMA,
        pltpu.SemaphoreType.DMA,
        pltpu.SemaphoreType.DMA,
        pltpu.SemaphoreType.DMA,
        pltpu.SemaphoreType.DMA,
        pltpu.SemaphoreType.DMA,
        pltpu.SemaphoreType.DMA,
        pltpu.SemaphoreType.DMA,
    ],
)
def _sc_aggregate(x_hbm, idx_hbm, emb_hbm, zero_hbm, out_hbm,
                  agg, idxv, dstc, xg, ev,
                  isem0, isem1, gsem0, gsem1, esem0, esem1, ssem0, ssem1):
    cid = lax.axis_index("c")
    sid = lax.axis_index("s")
    wid = sid * NC + cid
    e0 = wid * EPP
    g0 = wid * NCHUNK
    isem = (isem0, isem1)
    gsem = (gsem0, gsem1)
    esem = (esem0, esem1)
    ssem = (ssem0, ssem1)

    def _copy_dst_row(s):
        for j in range(C // 16):
            dstc[s, pl.ds(j * 16, 16)] = idxv[s, 1, pl.ds(j * 16, 16)]

    pltpu.sync_copy(idx_hbm.at[g0], idxv.at[0])
    pltpu.sync_copy(idx_hbm.at[g0 + 1], idxv.at[1])
    _copy_dst_row(0)
    _copy_dst_row(1)
    pltpu.async_copy(x_hbm.at[idxv.at[0, 0, pl.ds(0, C)]], xg.at[0], gsem[0])
    pltpu.async_copy(emb_hbm.at[pl.ds(e0, C), :], ev.at[0], esem[0])

    r0 = sid * ROWS_PER_TILE
    pltpu.sync_copy(zero_hbm.at[pl.ds(r0, ROWS_PER_TILE), :],
                    agg.at[pl.ds(r0, ROWS_PER_TILE), :])
    plsc.subcore_barrier()

    def _pair(jj, _):
        for b in (0, 1):
            k = 2 * jj + b
            nb = 1 - b
            @pl.when(k > 0)
            def _():
                pltpu.make_async_copy(
                    xg.at[nb], agg.at[dstc.at[nb]], ssem[nb]).wait()
            @pl.when(jnp.logical_and(k > 0, k + 1 < NCHUNK))
            def _():
                pltpu.make_async_copy(
                    idx_hbm.at[g0], idxv.at[nb], isem[nb]).wait()
            @pl.when(k + 1 < NCHUNK)
            def _():
                pltpu.async_copy(
                    x_hbm.at[idxv.at[nb, 0, pl.ds(0, C)]], xg.at[nb], gsem[nb])
                pltpu.async_copy(
                    emb_hbm.at[pl.ds(e0 + (k + 1) * C, C), :], ev.at[nb], esem[nb])
            @pl.when(jnp.logical_and(k > 0, k + 1 < NCHUNK))
            def _():
                _copy_dst_row(nb)
            @pl.when(k + 2 < NCHUNK)
            def _():
                pltpu.async_copy(idx_hbm.at[g0 + k + 2], idxv.at[b], isem[b])
            pltpu.make_async_copy(
                x_hbm.at[idxv.at[b, 0, pl.ds(0, C)]], xg.at[b], gsem[b]).wait()
            pltpu.make_async_copy(
                emb_hbm.at[pl.ds(e0, C), :], ev.at[b], esem[b]).wait()

            @plsc.parallel_loop(0, C, unroll=4)
            def _row(i):
                for j in range(D // 16):
                    sl = pl.ds(j * 16, 16)
                    xg[b, i, sl] = jnp.maximum(xg[b, i, sl] + ev[b, i, sl], 0.0)

            pltpu.async_copy(xg.at[b], agg.at[dstc.at[b]], ssem[b], add=True)
        return 0

    lax.fori_loop(0, NCHUNK // 2, _pair, 0)
    pltpu.make_async_copy(xg.at[1], agg.at[dstc.at[1]], ssem[1]).wait()
    plsc.subcore_barrier()

    pltpu.sync_copy(agg.at[pl.ds(r0, ROWS_PER_TILE), :],
                    out_hbm.at[cid, pl.ds(r0, ROWS_PER_TILE), :])


def _dense_bn_body(x_ref, p0_ref, p1_ref, w_ref, g_ref, b_ref, o_ref):
    y = x_ref[...] + p0_ref[0] + p1_ref[0]
    h = jnp.maximum(jnp.dot(y, w_ref[...], preferred_element_type=jnp.float32), 0.0)
    mu = jnp.mean(h, axis=0, keepdims=True)
    var = jnp.mean(h * h, axis=0, keepdims=True) - mu * mu
    inv = lax.rsqrt(var + EPS)
    o_ref[...] = (h - mu) * inv * g_ref[...] + b_ref[...] + x_ref[...]


def _dense_bn(x, partial, W, gamma, beta):
    x_spec = pl.BlockSpec((N, D), lambda i: (0, 0))
    p0_spec = pl.BlockSpec((1, N, D), lambda i: (0, 0, 0))
    p1_spec = pl.BlockSpec((1, N, D), lambda i: (1, 0, 0))
    w_spec = pl.BlockSpec((D, D), lambda i: (0, 0))
    row = pl.BlockSpec((1, D), lambda i: (0, 0))
    return pl.pallas_call(
        _dense_bn_body,
        grid=(1,),
        in_specs=[x_spec, p0_spec, p1_spec, w_spec, row, row],
        out_specs=x_spec,
        out_shape=jax.ShapeDtypeStruct((N, D), jnp.float32),
    )(x, partial, partial, W, gamma, beta)


def kernel(x, edge_index, edge_attr, batch,
           We0, W0, gamma0, beta0,
           We1, W1, gamma1, beta1,
           We2, W2, gamma2, beta2):
    del batch
    pad = EPP - EP
    src = jnp.pad(edge_index[0].reshape(NW, EP), ((0, 0), (0, pad)))
    dst = jnp.pad(edge_index[1].reshape(NW, EP), ((0, 0), (0, pad)),
                  constant_values=N)
    s3 = jnp.pad(src.reshape(NW, NCHUNK, C), ((0, 0), (0, 0), (0, 128 - C)))
    d3 = jnp.pad(dst.reshape(NW, NCHUNK, C), ((0, 0), (0, 0), (0, 128 - C)),
                 constant_values=N)
    idx = jnp.stack([s3, d3], axis=2).reshape(G, 2, 128)
    ea_pad = jnp.pad(edge_attr.reshape(NW, EP, DE),
                     ((0, 0), (0, pad), (0, 0))).reshape(-1, DE)
    zeros = jnp.zeros((NPAD, D), jnp.float32)
    embs = _edge_embeddings(ea_pad, We0, We1, We2)
    params = [(W0, gamma0, beta0), (W1, gamma1, beta1), (W2, gamma2, beta2)]
    for emb, (W, g, b) in zip(embs, params):
        partial = _sc_aggregate(x, idx, emb, zeros)
        x = _dense_bn(x, partial, W, g.reshape(1, D), b.reshape(1, D))
    return x
